# baseline (speedup 1.0000x reference)
import functools

import jax
import jax.numpy as jnp
from jax import lax
from jax.experimental import pallas as pl
from jax.experimental.pallas import tpu as pltpu
from jax.experimental.pallas import tpu_sc as plsc

EMBED_DIM = 64


@functools.lru_cache(maxsize=None)
def _make_gather(B: int, D: int):
    info = plsc.get_sparse_core_info()
    NC, NS = info.num_cores, info.num_subcores
    NW = NC * NS
    assert B % NW == 0
    b_per_w = B // NW
    CH = 1600
    n_ch = b_per_w // CH
    assert n_ch * CH == b_per_w

    mesh = plsc.VectorSubcoreMesh(core_axis_name="c", subcore_axis_name="s")

    @functools.partial(
        pl.kernel,
        mesh=mesh,
        out_type=jax.ShapeDtypeStruct((B, D), jnp.float32),
        scratch_types=[
            pltpu.VMEM((b_per_w,), jnp.int32),
            pltpu.VMEM((CH, D), jnp.float32),
            pltpu.SemaphoreType.DMA,
        ],
        compiler_params=pltpu.CompilerParams(use_tc_tiling_on_sc=False),
    )
    def gather_kernel(table_hbm, idx_hbm, out_hbm, idx_v, rows_v, sem):
        wid = lax.axis_index("s") * NC + lax.axis_index("c")
        base = wid * b_per_w
        pltpu.sync_copy(idx_hbm.at[pl.ds(base, b_per_w)], idx_v)
        for c in range(n_ch):
            pltpu.async_copy(
                table_hbm.at[idx_v.at[pl.ds(c * CH, CH)]], rows_v, sem
            ).wait()
            pltpu.sync_copy(rows_v, out_hbm.at[pl.ds(base + c * CH, CH)])

    return gather_kernel


def kernel(table, inputs):
    shape = inputs.shape
    idx = inputs.reshape(-1).astype(jnp.int32)
    out = _make_gather(idx.shape[0], table.shape[1])(table, idx)
    return out.reshape(*shape, table.shape[1])
